# baseline (speedup 1.0000x reference)
import functools

import jax
import jax.numpy as jnp
import numpy as np
from jax import lax
from jax.experimental import pallas as pl
from jax.experimental.pallas import tpu as pltpu
from jax.experimental.pallas import tpu_sc as plsc

D_MODEL = 64
MAXLEN = 200
VOCAB = 5
BATCH = 4096
ROWS = BATCH * MAXLEN
NC, NS = 2, 16
NW = NC * NS
RPW = ROWS // NW
BPW = BATCH // NW
CB = 4
CHUNK = CB * MAXLEN
NCHUNK = BPW // CB
JV = CHUNK // 16
NGATHER = 10
GR = CHUNK // NGATHER
L_PER = 13


def _positional_encoding() -> np.ndarray:
    pos = np.arange(MAXLEN)[:, None]
    i = np.arange(D_MODEL)[None, :]
    rates = 1 / np.power(10000, 2 * (i // 2) / np.float32(D_MODEL))
    angle = pos * rates
    angle[:, 0::2] = np.sin(angle[:, 0::2])
    angle[:, 1::2] = np.cos(angle[:, 1::2])
    return angle.astype(np.float32)


_PE = _positional_encoding()


def _body(tok_hbm, table_hbm, pe_hbm, out_hbm,
          pe_v, tab_v, build_v, fused_sh, pos5_v, tok_v, idx_v, rows_v, sem):
    s = lax.axis_index("s")
    c = lax.axis_index("c")
    wid = s * NC + c

    pltpu.sync_copy(table_hbm, tab_v)
    pltpu.sync_copy(pe_hbm, pe_v)
    for v in range(VOCAB):
        for k in range(D_MODEL // 16):
            tab_v[v, pl.ds(k * 16, 16)] = tab_v[v, pl.ds(k * 16, 16)] * 8.0
    l0 = s * L_PER
    for li in range(L_PER):
        l = l0 + li

        @pl.when(l < MAXLEN)
        def _build():
            for v in range(VOCAB):
                for k in range(D_MODEL // 16):
                    sl = pl.ds(k * 16, 16)
                    build_v[v, sl] = (tab_v[v, sl] + pe_v[l, sl]) * 0.125
            pltpu.sync_copy(build_v, fused_sh.at[pl.ds(l * VOCAB, VOCAB)])

    plsc.subcore_barrier()

    iota16 = lax.broadcasted_iota(jnp.int32, (16,), 0)
    for j in range(JV):
        pos5_v[pl.ds(j * 16, 16)] = lax.rem(j * 16 + iota16, MAXLEN) * VOCAB

    bat_w = wid * BPW

    @pl.loop(0, NCHUNK)
    def _chunk(g):
        b0 = bat_w + g * CB
        toks = []
        for k in range(CB):
            toks.append(pltpu.async_copy(
                tok_hbm.at[b0 + k], tok_v.at[pl.ds(k * MAXLEN, MAXLEN)], sem))
        for cp in toks:
            cp.wait()
        for j in range(JV):
            sl = pl.ds(j * 16, 16)
            idx_v[j // (GR // 16), pl.ds((j % (GR // 16)) * 16, 16)] = (
                pos5_v[sl] + tok_v[sl])
        copies = []
        for r in range(NGATHER):
            copies.append(pltpu.async_copy(
                fused_sh.at[idx_v.at[r]],
                rows_v.at[pl.ds(r * GR, GR)], sem))
        for cp in copies:
            cp.wait()
        outs = []
        for k in range(CB):
            outs.append(pltpu.async_copy(
                rows_v.at[pl.ds(k * MAXLEN, MAXLEN)], out_hbm.at[b0 + k], sem))
        for cp in outs:
            cp.wait()


@functools.partial(jax.jit, static_argnames=())
def _sc_embed(tok_flat, emb_table, pe):
    mesh = plsc.VectorSubcoreMesh(core_axis_name="c", subcore_axis_name="s",
                                  num_cores=NC, num_subcores=NS)
    return pl.kernel(
        _body,
        out_type=jax.ShapeDtypeStruct((BATCH, MAXLEN, D_MODEL), jnp.float32),
        mesh=mesh,
        scratch_types=[
            pltpu.VMEM((MAXLEN, D_MODEL), jnp.float32),
            pltpu.VMEM((VOCAB, D_MODEL), jnp.float32),
            pltpu.VMEM((VOCAB, D_MODEL), jnp.float32),
            pltpu.VMEM_SHARED((MAXLEN * VOCAB, D_MODEL), jnp.float32),
            pltpu.VMEM((CHUNK,), jnp.int32),
            pltpu.VMEM((CHUNK,), jnp.int32),
            pltpu.VMEM((NGATHER, GR), jnp.int32),
            pltpu.VMEM((CHUNK, D_MODEL), jnp.float32),
            pltpu.SemaphoreType.DMA,
        ],
        compiler_params=pltpu.CompilerParams(use_tc_tiling_on_sc=False),
    )(tok_flat, emb_table, pe)


def kernel(rnatok, emb_table):
    pe = jnp.asarray(_PE)
    return _sc_embed(rnatok, emb_table, pe)


# speedup vs baseline: 1.1120x; 1.1120x over previous
import functools

import jax
import jax.numpy as jnp
import numpy as np
from jax import lax
from jax.experimental import pallas as pl
from jax.experimental.pallas import tpu as pltpu
from jax.experimental.pallas import tpu_sc as plsc

D_MODEL = 64
MAXLEN = 200
VOCAB = 5
BATCH = 4096
ROWS = BATCH * MAXLEN
NC, NS = 2, 16
NW = NC * NS
RPW = ROWS // NW
CHUNK = 400
NCHUNK = RPW // CHUNK
JV = CHUNK // 16
NGATHER = 5
GR = CHUNK // NGATHER
L_PER = 13


def _positional_encoding() -> np.ndarray:
    pos = np.arange(MAXLEN)[:, None]
    i = np.arange(D_MODEL)[None, :]
    rates = 1 / np.power(10000, 2 * (i // 2) / np.float32(D_MODEL))
    angle = pos * rates
    angle[:, 0::2] = np.sin(angle[:, 0::2])
    angle[:, 1::2] = np.cos(angle[:, 1::2])
    return angle.astype(np.float32)


_PE = _positional_encoding()


def _body(tok_hbm, table_hbm, pe_hbm, out_hbm,
          pe_v, tab_v, build_v, fused_sh, pos5_v, tok_v, idx_v, rows_v,
          tsem, gsem, wsem0, wsem1):
    s = lax.axis_index("s")
    c = lax.axis_index("c")
    wid = s * NC + c

    pltpu.sync_copy(table_hbm, tab_v)
    pltpu.sync_copy(pe_hbm, pe_v)
    for v in range(VOCAB):
        for k in range(D_MODEL // 16):
            tab_v[v, pl.ds(k * 16, 16)] = tab_v[v, pl.ds(k * 16, 16)] * 8.0
    l0 = s * L_PER
    for li in range(L_PER):
        l = l0 + li

        @pl.when(l < MAXLEN)
        def _build():
            for v in range(VOCAB):
                for k in range(D_MODEL // 16):
                    sl = pl.ds(k * 16, 16)
                    build_v[v, sl] = (tab_v[v, sl] + pe_v[l, sl]) * 0.125
            pltpu.sync_copy(build_v, fused_sh.at[pl.ds(l * VOCAB, VOCAB)])

    plsc.subcore_barrier()

    iota16 = lax.broadcasted_iota(jnp.int32, (16,), 0)
    for j in range(JV):
        pos5_v[pl.ds(j * 16, 16)] = lax.rem(j * 16 + iota16, MAXLEN) * VOCAB

    row_w = wid * RPW
    wsems = (wsem0, wsem1)

    pltpu.async_copy(tok_hbm.at[pl.ds(row_w, CHUNK)], tok_v.at[0], tsem)

    @pl.loop(0, NCHUNK, step=2)
    def _chunk2(g0):
        for p in range(2):
            g = g0 + p
            row0 = row_w + g * CHUNK
            pltpu.make_async_copy(
                tok_hbm.at[pl.ds(row0, CHUNK)], tok_v.at[p], tsem).wait()

            @pl.when(g + 1 < NCHUNK)
            def _prefetch():
                pltpu.async_copy(
                    tok_hbm.at[pl.ds(row0 + CHUNK, CHUNK)],
                    tok_v.at[1 - p], tsem)

            for j in range(JV):
                sl = pl.ds(j * 16, 16)
                idx_v[p, j // (GR // 16), pl.ds((j % (GR // 16)) * 16, 16)] = (
                    pos5_v[sl] + tok_v[p, sl])

            @pl.when(g0 > 0)
            def _drain():
                pltpu.make_async_copy(
                    rows_v.at[p],
                    out_hbm.at[pl.ds(row0 - 2 * CHUNK, CHUNK)],
                    wsems[p]).wait()

            copies = []
            for r in range(NGATHER):
                copies.append(pltpu.async_copy(
                    fused_sh.at[idx_v.at[p, r]],
                    rows_v.at[p, pl.ds(r * GR, GR)], gsem))
            for cp in copies:
                cp.wait()
            pltpu.async_copy(
                rows_v.at[p], out_hbm.at[pl.ds(row0, CHUNK)], wsems[p])

    for p in range(2):
        row0 = row_w + (NCHUNK - 2 + p) * CHUNK
        pltpu.make_async_copy(
            rows_v.at[p], out_hbm.at[pl.ds(row0, CHUNK)], wsems[p]).wait()


@functools.partial(jax.jit, static_argnames=())
def _sc_embed(tok_flat, emb_table, pe):
    mesh = plsc.VectorSubcoreMesh(core_axis_name="c", subcore_axis_name="s",
                                  num_cores=NC, num_subcores=NS)
    out = pl.kernel(
        _body,
        out_type=jax.ShapeDtypeStruct((ROWS, D_MODEL), jnp.float32),
        mesh=mesh,
        scratch_types=[
            pltpu.VMEM((MAXLEN, D_MODEL), jnp.float32),
            pltpu.VMEM((VOCAB, D_MODEL), jnp.float32),
            pltpu.VMEM((VOCAB, D_MODEL), jnp.float32),
            pltpu.VMEM_SHARED((MAXLEN * VOCAB, D_MODEL), jnp.float32),
            pltpu.VMEM((CHUNK,), jnp.int32),
            pltpu.VMEM((2, CHUNK), jnp.int32),
            pltpu.VMEM((2, NGATHER, GR), jnp.int32),
            pltpu.VMEM((2, CHUNK, D_MODEL), jnp.float32),
            pltpu.SemaphoreType.DMA,
            pltpu.SemaphoreType.DMA,
            pltpu.SemaphoreType.DMA,
            pltpu.SemaphoreType.DMA,
        ],
        compiler_params=pltpu.CompilerParams(use_tc_tiling_on_sc=False),
    )(tok_flat, emb_table, pe)
    return out.reshape(BATCH, MAXLEN, D_MODEL)


def kernel(rnatok, emb_table):
    pe = jnp.asarray(_PE)
    return _sc_embed(rnatok.reshape(-1), emb_table, pe)
